# baseline (speedup 1.0000x reference)
import functools

import jax
import jax.numpy as jnp
from jax.experimental import pallas as pl
from jax.experimental.pallas import tpu as pltpu

_LANES = 128
_NEG = -1e30


def _moe_body(x_ref, rw_ref, rb_ref, w1_ref, b1_ref, w2_ref, b2_ref,
              out_ref, coef_ref, *, n_experts):
    e = pl.program_id(1)

    @pl.when(e == 0)
    def _router():
        logits = jnp.dot(x_ref[...], rw_ref[...],
                         preferred_element_type=jnp.float32) + rb_ref[...]
        lane = jax.lax.broadcasted_iota(jnp.int32, logits.shape, 1)
        m1 = jnp.max(logits, axis=1, keepdims=True)
        i1 = jnp.min(jnp.where(logits == m1, lane, _LANES), axis=1,
                     keepdims=True)
        l2 = jnp.where(lane == i1, _NEG, logits)
        m2 = jnp.max(l2, axis=1, keepdims=True)
        i2 = jnp.min(jnp.where(l2 == m2, lane, _LANES), axis=1,
                     keepdims=True)
        p0 = 1.0 / (1.0 + jnp.exp(m2 - m1))
        sel = (lane == i1) | (lane == i2)
        coef_ref[...] = jnp.where(sel, p0, 0.0)

    lane = jax.lax.broadcasted_iota(jnp.int32, coef_ref.shape, 1)
    col = jnp.sum(jnp.where(lane == e, coef_ref[...], 0.0), axis=1,
                  keepdims=True)
    h = jax.nn.gelu(jnp.dot(x_ref[...], w1_ref[0],
                            preferred_element_type=jnp.float32) + b1_ref[0])
    o = (jnp.dot(h, w2_ref[0], preferred_element_type=jnp.float32)
         + b2_ref[0]) * col

    @pl.when(e == 0)
    def _init():
        out_ref[...] = o

    @pl.when(e != 0)
    def _acc():
        out_ref[...] += o


def kernel(x, training, router_W, router_b, W1, b1, W2, b2):
    B, S, H = x.shape
    E = router_W.shape[1]
    F = W1.shape[2]
    xs = x.reshape(S, H)
    rwp = jnp.pad(router_W, ((0, 0), (0, _LANES - E)))
    rbp = jnp.concatenate(
        [router_b, jnp.full((_LANES - E,), _NEG, router_b.dtype)]
    ).reshape(1, _LANES)
    b1r = b1.reshape(E, 1, F)
    b2r = b2.reshape(E, 1, H)

    BS = 256
    grid = (S // BS, E)
    out = pl.pallas_call(
        functools.partial(_moe_body, n_experts=E),
        grid=grid,
        in_specs=[
            pl.BlockSpec((BS, H), lambda t, e: (t, 0)),
            pl.BlockSpec((H, _LANES), lambda t, e: (0, 0)),
            pl.BlockSpec((1, _LANES), lambda t, e: (0, 0)),
            pl.BlockSpec((1, H, F), lambda t, e: (e, 0, 0)),
            pl.BlockSpec((1, 1, F), lambda t, e: (e, 0, 0)),
            pl.BlockSpec((1, F, H), lambda t, e: (e, 0, 0)),
            pl.BlockSpec((1, 1, H), lambda t, e: (e, 0, 0)),
        ],
        out_specs=pl.BlockSpec((BS, H), lambda t, e: (t, 0)),
        out_shape=jax.ShapeDtypeStruct((S, H), jnp.float32),
        scratch_shapes=[pltpu.VMEM((BS, _LANES), jnp.float32)],
        compiler_params=pltpu.CompilerParams(
            dimension_semantics=("arbitrary", "arbitrary"),
        ),
    )(xs, rwp, rbp, W1, b1r, W2, b2r)
    return out.reshape(B, S, H)


# speedup vs baseline: 1.0838x; 1.0838x over previous
import functools

import jax
import jax.numpy as jnp
from jax.experimental import pallas as pl
from jax.experimental.pallas import tpu as pltpu

_LANES = 128
_NEG = -1e30


def _moe_body(x_ref, rw_ref, rb_ref, w1_ref, b1_ref, w2_ref, b2_ref,
              out_ref, coef_ref, *, n_experts):
    e = pl.program_id(1)

    @pl.when(e == 0)
    def _router():
        logits = jnp.dot(x_ref[...], rw_ref[...],
                         preferred_element_type=jnp.float32) + rb_ref[...]
        lane = jax.lax.broadcasted_iota(jnp.int32, logits.shape, 1)
        m1 = jnp.max(logits, axis=1, keepdims=True)
        i1 = jnp.min(jnp.where(logits == m1, lane, _LANES), axis=1,
                     keepdims=True)
        l2 = jnp.where(lane == i1, _NEG, logits)
        m2 = jnp.max(l2, axis=1, keepdims=True)
        i2 = jnp.min(jnp.where(l2 == m2, lane, _LANES), axis=1,
                     keepdims=True)
        p0 = 1.0 / (1.0 + jnp.exp(m2 - m1))
        sel = (lane == i1) | (lane == i2)
        coef_ref[...] = jnp.where(sel, p0, 0.0)

    lane = jax.lax.broadcasted_iota(jnp.int32, coef_ref.shape, 1)
    col = jnp.sum(jnp.where(lane == e, coef_ref[...], 0.0), axis=1,
                  keepdims=True)
    xb = x_ref[...].astype(jnp.bfloat16)
    h = jax.nn.gelu(jnp.dot(xb, w1_ref[0],
                            preferred_element_type=jnp.float32) + b1_ref[0])
    o = (jnp.dot(h.astype(jnp.bfloat16), w2_ref[0],
                 preferred_element_type=jnp.float32)
         + b2_ref[0]) * col

    @pl.when(e == 0)
    def _init():
        out_ref[...] = o

    @pl.when(e != 0)
    def _acc():
        out_ref[...] += o


def kernel(x, training, router_W, router_b, W1, b1, W2, b2):
    B, S, H = x.shape
    E = router_W.shape[1]
    F = W1.shape[2]
    xs = x.reshape(S, H)
    rwp = jnp.pad(router_W, ((0, 0), (0, _LANES - E)))
    rbp = jnp.concatenate(
        [router_b, jnp.full((_LANES - E,), _NEG, router_b.dtype)]
    ).reshape(1, _LANES)
    b1r = b1.reshape(E, 1, F)
    b2r = b2.reshape(E, 1, H)
    W1b = W1.astype(jnp.bfloat16)
    W2b = W2.astype(jnp.bfloat16)

    BS = 256
    grid = (S // BS, E)
    out = pl.pallas_call(
        functools.partial(_moe_body, n_experts=E),
        grid=grid,
        in_specs=[
            pl.BlockSpec((BS, H), lambda t, e: (t, 0)),
            pl.BlockSpec((H, _LANES), lambda t, e: (0, 0)),
            pl.BlockSpec((1, _LANES), lambda t, e: (0, 0)),
            pl.BlockSpec((1, H, F), lambda t, e: (e, 0, 0)),
            pl.BlockSpec((1, 1, F), lambda t, e: (e, 0, 0)),
            pl.BlockSpec((1, F, H), lambda t, e: (e, 0, 0)),
            pl.BlockSpec((1, 1, H), lambda t, e: (e, 0, 0)),
        ],
        out_specs=pl.BlockSpec((BS, H), lambda t, e: (t, 0)),
        out_shape=jax.ShapeDtypeStruct((S, H), jnp.float32),
        scratch_shapes=[pltpu.VMEM((BS, _LANES), jnp.float32)],
        compiler_params=pltpu.CompilerParams(
            dimension_semantics=("arbitrary", "arbitrary"),
        ),
    )(xs, rwp, rbp, W1b, b1r, W2b, b2r)
    return out.reshape(B, S, H)


# speedup vs baseline: 1.7234x; 1.5902x over previous
import functools

import jax
import jax.numpy as jnp
from jax.experimental import pallas as pl
from jax.experimental.pallas import tpu as pltpu

_LANES = 128
_NEG = -1e30


def _moe_body(x_ref, rw_ref, rb_ref, w1_ref, b1_ref, w2_ref, b2_ref,
              out_ref, *, n_experts):
    logits = jnp.dot(x_ref[...], rw_ref[...],
                     preferred_element_type=jnp.float32) + rb_ref[...]
    lane = jax.lax.broadcasted_iota(jnp.int32, logits.shape, 1)
    m1 = jnp.max(logits, axis=1, keepdims=True)
    i1 = jnp.min(jnp.where(logits == m1, lane, _LANES), axis=1, keepdims=True)
    l2 = jnp.where(lane == i1, _NEG, logits)
    m2 = jnp.max(l2, axis=1, keepdims=True)
    i2 = jnp.min(jnp.where(l2 == m2, lane, _LANES), axis=1, keepdims=True)
    p0 = 1.0 / (1.0 + jnp.exp(m2 - m1))
    sel = (lane == i1) | (lane == i2)
    coef = jnp.where(sel, p0, 0.0)

    xb = x_ref[...].astype(jnp.bfloat16)
    acc = None
    for e in range(n_experts):
        col = coef[:, e:e + 1]
        h = jax.nn.gelu(
            jnp.dot(xb, w1_ref[e], preferred_element_type=jnp.float32)
            + b1_ref[e])
        o = (jnp.dot(h.astype(jnp.bfloat16), w2_ref[e],
                     preferred_element_type=jnp.float32) + b2_ref[e]) * col
        acc = o if acc is None else acc + o
    out_ref[...] = acc


def kernel(x, training, router_W, router_b, W1, b1, W2, b2):
    B, S, H = x.shape
    E = router_W.shape[1]
    F = W1.shape[2]
    xs = x.reshape(S, H)
    rwp = jnp.pad(router_W, ((0, 0), (0, _LANES - E)))
    rbp = jnp.concatenate(
        [router_b, jnp.full((_LANES - E,), _NEG, router_b.dtype)]
    ).reshape(1, _LANES)
    W1b = W1.astype(jnp.bfloat16)
    W2b = W2.astype(jnp.bfloat16)

    BS = 256
    grid = (S // BS,)
    out = pl.pallas_call(
        functools.partial(_moe_body, n_experts=E),
        grid=grid,
        in_specs=[
            pl.BlockSpec((BS, H), lambda t: (t, 0)),
            pl.BlockSpec((H, _LANES), lambda t: (0, 0)),
            pl.BlockSpec((1, _LANES), lambda t: (0, 0)),
            pl.BlockSpec((E, H, F), lambda t: (0, 0, 0)),
            pl.BlockSpec((E, F), lambda t: (0, 0)),
            pl.BlockSpec((E, F, H), lambda t: (0, 0, 0)),
            pl.BlockSpec((E, H), lambda t: (0, 0)),
        ],
        out_specs=pl.BlockSpec((BS, H), lambda t: (t, 0)),
        out_shape=jax.ShapeDtypeStruct((S, H), jnp.float32),
        compiler_params=pltpu.CompilerParams(
            dimension_semantics=("arbitrary",),
        ),
    )(xs, rwp, rbp, W1b, b1, W2b, b2)
    return out.reshape(B, S, H)
